# baseline (speedup 1.0000x reference)
import numpy as np
import jax
import jax.numpy as jnp
from jax.experimental import pallas as pl
from jax.experimental.pallas import tpu as pltpu

_SMOOTH_EPS = 0.001
_NA = 4
_R = 512
_EPB = 64 * _R

_PK = []
for _k in range(2):
    _p = np.zeros((64, 32), np.float32)
    _p[np.arange(32) + 32 * _k, np.arange(32)] = 1.0
    _PK.append(_p)
_QJ = []
for _j in range(4):
    _q = np.zeros((32, 8), np.float32)
    _q[np.arange(8) + 8 * _j, np.arange(8)] = 1.0
    _QJ.append(_q)
_I8 = np.eye(8, dtype=np.float32)
_GRP = np.kron(np.eye(64, dtype=np.float32), np.ones((4, 4), np.float32))


def _build_tables(slab):
    w1 = slab[0:8, 0:2].T
    b1 = slab[0:8, 32]
    w2 = slab[8:40, 0:8].T
    b2 = slab[8:40, 32]
    w3 = slab[40:72, 0:32].T
    b3 = slab[40:72, 32]
    w4 = slab[72:76, 0:32].T
    b4 = slab[72:76, 32]

    w1h = w1.astype(jnp.bfloat16).astype(jnp.float32)
    w1l = w1 - w1h

    w1s = jnp.stack([jnp.kron(_PK[0], w1h), jnp.kron(_PK[1], w1h),
                     jnp.kron(_PK[0], w1l), jnp.kron(_PK[1], w1l)], 0)

    mats = []
    for j in range(4):
        mats.append(jnp.kron(_QJ[j], w2))
    mats.append(jnp.kron(_I8, w3))
    k4 = jnp.kron(_I8, w4)
    for k in range(2):
        for j in range(4):
            m = 128 * k + 32 * j
            mats.append(jnp.zeros((256, 256), jnp.float32)
                        .at[:, m:m + 32].set(k4))
    mats.append(jnp.asarray(_GRP))
    wstack = jnp.stack(mats, axis=0)

    btab = jnp.zeros((8, 256), jnp.float32)
    btab = btab.at[0].set(jnp.tile(b1, 32))
    btab = btab.at[1].set(jnp.tile(b2, 8))
    btab = btab.at[2].set(jnp.tile(b3, 8))
    btab = btab.at[3].set(jnp.tile(b4, 64))
    return w1s, wstack, btab


def _fp_kernel(x_ref, w1_ref, w_ref, b_ref, o_ref):
    f32 = jnp.float32
    dot = lambda a, b: jnp.dot(a, b, preferred_element_type=f32)

    x = x_ref[...]
    xh = x.astype(jnp.bfloat16).astype(f32)
    xl = x - xh

    h1s = []
    for k in range(2):
        t = (dot(xh, w1_ref[k]) + dot(xl, w1_ref[k])
             + dot(xh, w1_ref[2 + k]))
        h1s.append(jnp.maximum(t + b_ref[0], 0.0))
    h1 = jnp.concatenate(h1s, 0)

    h2s = [jnp.maximum(dot(h1, w_ref[j]) + b_ref[1], 0.0)
           for j in range(4)]
    h2 = jnp.concatenate(h2s, 0)

    h3 = jnp.maximum(dot(h2, w_ref[4]) + b_ref[2], 0.0)

    z = None
    for j in range(4):
        for k in range(2):
            t = h3[2 * _R * j + _R * k: 2 * _R * j + _R * (k + 1)]
            r = dot(t, w_ref[5 + 4 * k + j])
            z = r if z is None else z + r
    z = z + b_ref[3]

    c = jnp.max(z, axis=1, keepdims=True)
    e = jnp.exp(jnp.maximum(z - c, -60.0))
    s = dot(e, w_ref[13])
    cs = 1.0 / (1.0 + _SMOOTH_EPS * _NA)
    o_ref[...] = e * (pl.reciprocal(s, approx=True) * cs) + _SMOOTH_EPS * cs


def kernel(x, slab):
    B = x.shape[0]
    Bp = ((B + _EPB - 1) // _EPB) * _EPB
    if Bp != B:
        x = jnp.pad(x, ((0, Bp - B), (0, 0)))
    x2 = x.reshape(Bp // 64, 128)
    w1s, wstack, btab = _build_tables(slab)

    grid = Bp // _EPB
    out = pl.pallas_call(
        _fp_kernel,
        grid=(grid,),
        in_specs=[
            pl.BlockSpec((_R, 128), lambda i: (i, 0)),
            pl.BlockSpec((4, 128, 256), lambda i: (0, 0, 0)),
            pl.BlockSpec((14, 256, 256), lambda i: (0, 0, 0)),
            pl.BlockSpec((8, 256), lambda i: (0, 0)),
        ],
        out_specs=pl.BlockSpec((_R, 256), lambda i: (i, 0)),
        out_shape=jax.ShapeDtypeStruct((Bp // 64, 256), jnp.float32),
        compiler_params=pltpu.CompilerParams(
            dimension_semantics=("parallel",)),
        cost_estimate=pl.CostEstimate(
            flops=2 * Bp * 1424, transcendentals=Bp * 5,
            bytes_accessed=4 * (2 * Bp + 4 * Bp)),
    )(x2, w1s, wstack, btab)
    out = out.reshape(Bp, 4)
    return out[:B] if Bp != B else out
